# baseline (speedup 1.0000x reference)
import functools

import jax
import jax.numpy as jnp
from jax import lax
from jax.experimental import pallas as pl
from jax.experimental.pallas import tpu as pltpu
from jax.experimental.pallas import tpu_sc as plsc

NC = 2
NS = 16
NW = NC * NS
L = 16
QC = 4
CT_PER_W = 4


def _gather_transposed(table_t, flat_idx, n, d):
    n_ct = d // 8
    n_jt = n // 128
    n_pairs = n_jt // (2 * QC)
    mesh = plsc.VectorSubcoreMesh(
        core_axis_name="c", subcore_axis_name="s",
        num_cores=NC, num_subcores=NS)

    @functools.partial(
        pl.kernel,
        mesh=mesh,
        compiler_params=pltpu.CompilerParams(use_tc_tiling_on_sc=False, needs_layout_passes=False),
        out_type=jax.ShapeDtypeStruct((n_ct, n_jt, 8, 128), jnp.float32),
        scratch_types=[
            pltpu.VMEM((n,), jnp.int32),
            pltpu.VMEM((8, d), jnp.float32),
            pltpu.VMEM((QC, 8, 128), jnp.float32),
            pltpu.VMEM((QC, 8, 128), jnp.float32),
            pltpu.SemaphoreType.DMA,
            pltpu.SemaphoreType.DMA,
        ],
    )
    def run(tab_hbm, idx_hbm, out_hbm, idx_v, slab, buf0, buf1, sw0, sw1):
        wid = lax.axis_index("s") * NC + lax.axis_index("c")
        pltpu.sync_copy(idx_hbm, idx_v)

        def build(buf, q0):
            nb = 4
            units = [(kj, l4) for kj in range(QC)
                     for l4 in range(128 // L // nb)]

            def gathers(kj, l4):
                ls = [nb * l4 + i for i in range(nb)]
                ivs = [idx_v[pl.ds((q0 + kj) * 128 + l * L, L)]
                       for l in ls]
                return ls, [[plsc.load_gather(slab.at[c], [iv])
                             for c in range(8)] for iv in ivs]

            def stores(kj, ls, vals):
                for i, l in enumerate(ls):
                    for c in range(8):
                        buf[kj, c, pl.ds(l * L, L)] = vals[i][c]

            prev = None
            for kj, l4 in units:
                cur = (kj,) + gathers(kj, l4)
                if prev is not None:
                    stores(*prev)
                prev = cur
            stores(*prev)

        def start_write(buf, ct, q0, sem):
            return pltpu.async_copy(
                buf, out_hbm.at[ct].at[pl.ds(q0, QC)], sem)

        def wait_write(buf, sem):
            pltpu.make_async_copy(
                buf, out_hbm.at[0].at[pl.ds(0, QC)], sem).wait()

        for k in range(CT_PER_W):
            ct = wid * CT_PER_W + k

            @pl.when(ct < n_ct)
            def _():
                pltpu.sync_copy(tab_hbm.at[pl.ds(ct * 8, 8)], slab)

                def body(g, carry):
                    @pl.when(g > 0)
                    def _():
                        wait_write(buf0, sw0)
                    build(buf0, (2 * g) * QC)
                    start_write(buf0, ct, (2 * g) * QC, sw0)

                    @pl.when(g > 0)
                    def _():
                        wait_write(buf1, sw1)
                    build(buf1, (2 * g + 1) * QC)
                    start_write(buf1, ct, (2 * g + 1) * QC, sw1)
                    return carry

                lax.fori_loop(0, n_pairs, body, 0)
                wait_write(buf0, sw0)
                wait_write(buf1, sw1)

    return run(table_t, flat_idx)


def kernel(idx, targets, token_embedding_table):
    del targets
    b, t = idx.shape
    n = b * t
    v, d = token_embedding_table.shape
    flat_idx = idx.reshape(n).astype(jnp.int32)
    table_t = token_embedding_table.T
    out4d = _gather_transposed(table_t, flat_idx, n, d)
    return out4d.transpose(1, 3, 0, 2).reshape(n, d)


# speedup vs baseline: 1.0217x; 1.0217x over previous
import functools

import jax
import jax.numpy as jnp
from jax import lax
from jax.experimental import pallas as pl
from jax.experimental.pallas import tpu as pltpu
from jax.experimental.pallas import tpu_sc as plsc

NC = 2
NS = 16
NW = NC * NS
L = 16
QC = 4
CT_PER_W = 4


def _gather_transposed(table_t, flat_idx, n, d):
    n_ct = d // 8
    n_jt = n // 128
    n_pairs = n_jt // (2 * QC)
    mesh = plsc.VectorSubcoreMesh(
        core_axis_name="c", subcore_axis_name="s",
        num_cores=NC, num_subcores=NS)

    @functools.partial(
        pl.kernel,
        mesh=mesh,
        compiler_params=pltpu.CompilerParams(use_tc_tiling_on_sc=False, needs_layout_passes=False),
        out_type=jax.ShapeDtypeStruct((n_ct, n_jt, 8, 128), jnp.float32),
        scratch_types=[
            pltpu.VMEM((n,), jnp.int32),
            pltpu.VMEM((8, d), jnp.float32),
            pltpu.VMEM((QC, 8, 128), jnp.float32),
            pltpu.VMEM((QC, 8, 128), jnp.float32),
            pltpu.SemaphoreType.DMA,
            pltpu.SemaphoreType.DMA,
        ],
    )
    def run(tab_hbm, idx_hbm, out_hbm, idx_v, slab, buf0, buf1, sw0, sw1):
        wid = lax.axis_index("s") * NC + lax.axis_index("c")
        pltpu.sync_copy(idx_hbm, idx_v)

        def build(buf, q0):
            nb = 8
            for kj in range(QC):
                for l4 in range(128 // L // nb):
                    ls = [nb * l4 + i for i in range(nb)]
                    ivs = [idx_v[pl.ds((q0 + kj) * 128 + l * L, L)]
                           for l in ls]
                    vals = [[plsc.load_gather(slab.at[c], [iv])
                             for c in range(8)] for iv in ivs]
                    for i, l in enumerate(ls):
                        for c in range(8):
                            buf[kj, c, pl.ds(l * L, L)] = vals[i][c]

        def start_write(buf, ct, q0, sem):
            return pltpu.async_copy(
                buf, out_hbm.at[ct].at[pl.ds(q0, QC)], sem)

        def wait_write(buf, sem):
            pltpu.make_async_copy(
                buf, out_hbm.at[0].at[pl.ds(0, QC)], sem).wait()

        for k in range(CT_PER_W):
            ct = wid * CT_PER_W + k

            @pl.when(ct < n_ct)
            def _():
                pltpu.sync_copy(tab_hbm.at[pl.ds(ct * 8, 8)], slab)

                def body(g, carry):
                    @pl.when(g > 0)
                    def _():
                        wait_write(buf0, sw0)
                    build(buf0, (2 * g) * QC)
                    start_write(buf0, ct, (2 * g) * QC, sw0)

                    @pl.when(g > 0)
                    def _():
                        wait_write(buf1, sw1)
                    build(buf1, (2 * g + 1) * QC)
                    start_write(buf1, ct, (2 * g + 1) * QC, sw1)
                    return carry

                lax.fori_loop(0, n_pairs, body, 0)
                wait_write(buf0, sw0)
                wait_write(buf1, sw1)

    return run(table_t, flat_idx)


def kernel(idx, targets, token_embedding_table):
    del targets
    b, t = idx.shape
    n = b * t
    v, d = token_embedding_table.shape
    flat_idx = idx.reshape(n).astype(jnp.int32)
    table_t = token_embedding_table.T
    out4d = _gather_transposed(table_t, flat_idx, n, d)
    return out4d.transpose(1, 3, 0, 2).reshape(n, d)
